# baseline (speedup 1.0000x reference)
import functools

import jax
import jax.numpy as jnp
from jax import lax
from jax.experimental import pallas as pl
from jax.experimental.pallas import tpu as pltpu
from jax.experimental.pallas import tpu_sc as plsc

_LANES = 16
_CC = 16


@functools.lru_cache(maxsize=None)
def _make_sc_add(B: int, S: int, D: int):
    info = plsc.get_sparse_core_info()
    NC, NS = info.num_cores, info.num_subcores
    NW = NC * NS

    rows_per_w = S // NW
    T_ROWS = 8
    n_tiles = rows_per_w // T_ROWS
    n_groups = n_tiles // 2
    n_cc = D // (_LANES * _CC)
    assert S % NW == 0 and rows_per_w % T_ROWS == 0
    assert D % (_LANES * _CC) == 0 and n_tiles % 2 == 0 and n_groups >= 2

    mesh = plsc.VectorSubcoreMesh(core_axis_name="c", subcore_axis_name="s")

    @functools.partial(
        pl.kernel,
        mesh=mesh,
        out_type=jax.ShapeDtypeStruct((B * S, D), jnp.float32),
        scratch_types=(
            [pltpu.VMEM((T_ROWS, D), jnp.float32)] * 8
            + [pltpu.VMEM((T_ROWS, D), jnp.float32)] * 2
            + [pltpu.SemaphoreType.DMA] * 8
            + [pltpu.SemaphoreType.DMA] * 8
            + [pltpu.SemaphoreType.DMA] * 2
        ),
    )
    def k(x_hbm, pe_hbm, out_hbm, *rest):
        xb = rest[0:8]
        peb = rest[8:10]
        ls = rest[10:18]
        ss = rest[18:26]
        pes = rest[26:28]
        wid = lax.axis_index("s") * NC + lax.axis_index("c")
        w_row = wid * rows_per_w

        def x_rows(t, b):
            return pl.ds(b * S + w_row + t * T_ROWS, T_ROWS)

        def pe_load(t, pj):
            return pltpu.async_copy(
                pe_hbm.at[pl.ds(w_row + t * T_ROWS, T_ROWS)], peb[pj],
                pes[pj])

        def load(t, b, slot):
            return pltpu.async_copy(x_hbm.at[x_rows(t, b)], xb[slot],
                                    ls[slot])

        def store(t, b, slot):
            return pltpu.async_copy(xb[slot], out_hbm.at[x_rows(t, b)],
                                    ss[slot])

        def add_tile(slots, pj):
            bufs = [xb[s] for s in slots]
            pb = peb[pj]

            @plsc.parallel_loop(0, T_ROWS, unroll=1)
            def add_rows(r):
                @plsc.parallel_loop(0, n_cc, unroll=1)
                def add_cols(cc):
                    base = cc * (_LANES * _CC)
                    sls = [pl.ds(base + c * _LANES, _LANES)
                           for c in range(_CC)]
                    pvals = [pb[r, sl] for sl in sls]
                    for buf in bufs:
                        for sl, pv in zip(sls, pvals):
                            buf[r, sl] = buf[r, sl] + pv

        def group_body(g, drain):
            for j in range(8):
                tt, b = divmod(j, B)
                t = 2 * g + tt
                if drain:
                    pltpu.make_async_copy(
                        xb[j], out_hbm.at[x_rows(t, b)], ss[j]).wait()
                load(t, b, j)
            for tt in range(2):
                t = 2 * g + tt
                slots = [tt * B + b for b in range(B)]
                for j in slots:
                    pltpu.make_async_copy(
                        x_hbm.at[x_rows(t, j % B)], xb[j], ls[j]).wait()
                pltpu.make_async_copy(
                    pe_hbm.at[pl.ds(w_row, T_ROWS)], peb[tt],
                    pes[tt]).wait()
                add_tile(slots, tt)
                for j in slots:
                    store(t, j % B, j)

                @pl.when(g < n_groups - 1)
                def _():
                    pe_load(2 * (g + 1) + tt, tt)

        pe_load(0, 0)
        pe_load(1, 1)
        group_body(0, drain=False)
        lax.fori_loop(1, n_groups, lambda g, c: (group_body(g, True), c)[1],
                      0)
        for j in range(8):
            tt, b = divmod(j, B)
            pltpu.make_async_copy(
                xb[j], out_hbm.at[x_rows(2 * (n_groups - 1) + tt, b)],
                ss[j]).wait()

    return k


def kernel(x, pe_emb):
    B, S, D = x.shape
    k = _make_sc_add(B, S, D)
    out = k(x.reshape(B * S, D), pe_emb)
    return out.reshape(B, S, D)
